# Initial kernel scaffold; baseline (speedup 1.0000x reference)
import jax
import jax.numpy as jnp
from jax.experimental import pallas as pl


def kernel(user_embedding, item_embedding, social_vals, ui_vals, iu_vals, low_W1, low_b1, low_W2, low_b2, ga_W1, ga_b1, ga_W2, ga_b2, social_idx, ui_idx, iu_idx):
    raise NotImplementedError("write your pallas kernel here")


# speedup vs baseline: 37.3792x; 37.3792x over previous
import functools

import jax
import jax.numpy as jnp
from jax import lax
from jax.experimental import pallas as pl
from jax.experimental.pallas import tpu as pltpu
from jax.experimental.pallas import tpu_sc as plsc

N = 50000
D = 64
E = 800000
NC, NS = 2, 16
SPAD = 50176
SHALF = SPAD // 2
HALF = 25000
ACC_ROWS = 25088
TRASH = 25000
K = 400
SUB = 200
NSUB = K // SUB
EPT = E // NS
CHUNKS = EPT // K

_mesh = plsc.VectorSubcoreMesh(
    core_axis_name="c", subcore_axis_name="s", num_cores=NC, num_subcores=NS)


def _bcast16(ref, k):
  return plsc.load_gather(ref, [jnp.full((16,), k, jnp.int32)])


def _edge_exp(v, w1, b1, w2, b2):
  x = jnp.clip(v * w1 + b1, -30.0, 30.0)
  h = 1.0 / (1.0 + jnp.exp(-x))
  a = h * w2 + b2
  a = jnp.maximum(a, 0.2 * a)
  a = jnp.minimum(a, 60.0)
  return jnp.exp(a)


DH = D // 2


@functools.partial(
    pl.kernel,
    out_type=(jax.ShapeDtypeStruct((2, N, DH), jnp.float32),
              jax.ShapeDtypeStruct((N,), jnp.float32)),
    mesh=_mesh,
    compiler_params=pltpu.CompilerParams(needs_layout_passes=False,
                                         use_tc_tiling_on_sc=False),
    scratch_types=dict(
        acc=pltpu.VMEM_SHARED((SPAD, DH), jnp.float32),
        sacc=pltpu.VMEM_SHARED((SPAD,), jnp.float32),
        ebuf=pltpu.VMEM((2, 3 * K), jnp.int32),
        wv=pltpu.VMEM((K,), jnp.float32),
        dloc=pltpu.VMEM((2, NSUB, SUB), jnp.int32),
        rows=pltpu.VMEM((2, SUB, DH), jnp.float32),
        lpv=pltpu.VMEM((8,), jnp.float32),
        zb=pltpu.VMEM((16, DH), jnp.float32),
        zb1=pltpu.VMEM((224,), jnp.float32),
        esem=pltpu.SemaphoreType.DMA,
        gsem0=pltpu.SemaphoreType.DMA,
        gsem1=pltpu.SemaphoreType.DMA,
        ssem0=pltpu.SemaphoreType.DMA,
        ssem1=pltpu.SemaphoreType.DMA,
        tsem0=pltpu.SemaphoreType.DMA,
        tsem1=pltpu.SemaphoreType.DMA,
    ),
)
def _spmm(emb2, edata, lp, out, out_s,
          acc, sacc, ebuf, wv, dloc, rows, lpv, zb, zb1,
          esem, gsem0, gsem1, ssem0, ssem1, tsem0, tsem1):
  sid = lax.axis_index("s")
  c = lax.axis_index("c")
  gsems = (gsem0, gsem1)
  ssems = (ssem0, ssem1)
  tsems = (tsem0, tsem1)
  pltpu.sync_copy(lp, lpv)
  w1 = _bcast16(lpv, 0)
  b1 = _bcast16(lpv, 1)
  w2 = _bcast16(lpv, 2)
  b2 = _bcast16(lpv, 3)
  z16 = jnp.zeros((16,), jnp.float32)
  for r in range(16):
    for cc in range(2):
      zb[r, pl.ds(cc * 16, 16)] = z16
  for t in range(14):
    zb1[pl.ds(t * 16, 16)] = z16
  for t in range(196):
    pltpu.sync_copy(zb, acc.at[pl.ds(sid * 3136 + t * 16, 16)])
  for t in range(14):
    pltpu.sync_copy(zb1, sacc.at[pl.ds(sid * 3136 + t * 224, 224)])
  plsc.subcore_barrier()

  myemb = emb2.at[c]

  def fire_edges(i_next, par):
    off = pl.multiple_of((sid * CHUNKS + i_next) * (3 * K), 8)
    pltpu.async_copy(edata.at[pl.ds(off, 3 * K)], ebuf.at[par], esem)
    for b in range(NSUB):
      pltpu.async_copy(edata.at[pl.ds(off + K + b * SUB, SUB)],
                       dloc.at[par, b], esem)

  def drain_edges(par):
    pltpu.make_async_copy(edata.at[pl.ds(0, 3 * K)], ebuf.at[par],
                          esem).wait()
    for b in range(NSUB):
      pltpu.make_async_copy(edata.at[pl.ds(0, SUB)], dloc.at[par, b],
                            esem).wait()

  def fire_gather(eb, b):
    pltpu.async_copy(myemb.at[eb.at[pl.ds(b * SUB, SUB)]], rows.at[b],
                     gsems[b])

  def drain_gather(b):
    pltpu.make_async_copy(myemb.at[pl.ds(0, SUB)], rows.at[b],
                          gsems[b]).wait()

  def do_chunk(par, prefetch_i, fire_next):
    eb = ebuf.at[par]
    fire_gather(eb, 1)
    if prefetch_i is not None:
      fire_edges(prefetch_i, 1 - par)
    for j in range(K // 16):
      t = j * 16
      v = plsc.bitcast(eb[pl.ds(2 * K + t, 16)], jnp.float32)
      wv[pl.ds(t, 16)] = _edge_exp(v, w1, b1, w2, b2)
    scat = []
    for b in range(NSUB):
      drain_gather(b)

      @plsc.parallel_loop(0, SUB, step=1, unroll=8)
      def _(j, b=b):
        wb = plsc.load_gather(wv, [jnp.full((16,), b * SUB + j, jnp.int32)])
        for cc in range(2):
          rows[b, j, pl.ds(cc * 16, 16)] = (
              rows[b, j, pl.ds(cc * 16, 16)] * wb)

      scat.append(pltpu.async_copy(rows.at[b], acc.at[dloc.at[par, b]],
                                   ssems[b], add=True))
      scat.append(pltpu.async_copy(wv.at[pl.ds(b * SUB, SUB)],
                                   sacc.at[dloc.at[par, b]],
                                   tsems[b], add=True))
    scat[0].wait()
    scat[1].wait()
    if fire_next:
      drain_edges(1 - par)
      fire_gather(ebuf.at[1 - par], 0)
    scat[2].wait()
    scat[3].wait()

  fire_edges(0, 0)
  drain_edges(0)
  fire_gather(ebuf.at[0], 0)

  def pair(i, _):
    do_chunk(0, 2 * i + 1, True)
    do_chunk(1, 2 * i + 2, True)
    return 0

  lax.fori_loop(0, (CHUNKS - 1) // 2, pair, 0)
  do_chunk(0, None, False)
  plsc.subcore_barrier()

  myout = out.at[c]

  @pl.when(sid < 15)
  def _():
    r0 = sid * 3128
    pltpu.sync_copy(acc.at[pl.ds(r0, 3128)],
                    myout.at[pl.ds(pl.multiple_of(r0, 8), 3128)])

  @pl.when(sid == 15)
  def _():
    pltpu.sync_copy(acc.at[pl.ds(46920, 3080)],
                    myout.at[pl.ds(pl.multiple_of(46920, 8), 3080)])

  @pl.when((sid == 0) & (c == 0))
  def _():
    pltpu.sync_copy(sacc.at[pl.ds(0, N)], out_s.at[pl.ds(0, N)])


_BR = 1024


def _att(x, w1row, b1, w2, b2):
  h = jnp.tanh(jnp.sum(x * w1row, axis=1, keepdims=True) + b1)
  a = h * w2 + b2
  return jnp.maximum(a, 0.2 * a)


def _softmax_blend(ja, jb, s, t, w1_ref, sc_ref):
  a_s = _att(s, w1_ref[ja, :][None, :], sc_ref[ja, 0], sc_ref[ja, 1],
             sc_ref[ja, 2])
  a_t = _att(t, w1_ref[jb, :][None, :], sc_ref[jb, 0], sc_ref[jb, 1],
             sc_ref[jb, 2])
  m = jnp.maximum(a_s, a_t)
  es = jnp.exp(a_s - m)
  et = jnp.exp(a_t - m)
  return (es * s + et * t) / (es + et)


def _unsplit(x2_ref):
  return jnp.concatenate([x2_ref[0], x2_ref[1]], axis=-1)


def _write_both(out_ref, out2_ref, x):
  out_ref[...] = x
  out2_ref[0] = x[:, :DH]
  out2_ref[1] = x[:, DH:]


def _blend_u_body(ja, jb, base_ref, s_ref, t_ref, ss_ref, st_ref,
                  w1_ref, sc_ref, out_ref, out2_ref):
  s = _unsplit(s_ref) / (ss_ref[...] + 1e-10)
  t = _unsplit(t_ref) / (st_ref[...] + 1e-10)
  blended = _softmax_blend(ja, jb, s, t, w1_ref, sc_ref)
  _write_both(out_ref, out2_ref, 0.5 * base_ref[...] + 0.5 * blended)


def _blend_i_body(ja, jb, base_ref, t_ref, st_ref, w1_ref, sc_ref,
                  out_ref, out2_ref):
  t = _unsplit(t_ref) / (st_ref[...] + 1e-10)
  blended = _softmax_blend(ja, jb, base_ref[...], t, w1_ref, sc_ref)
  _write_both(out_ref, out2_ref, blended)


_ROWS_SPEC = pl.BlockSpec((_BR, D), lambda i: (i, 0))
_SPLIT_SPEC = pl.BlockSpec((2, _BR, DH), lambda i: (0, i, 0))
_SEG_SPEC = pl.BlockSpec((_BR, 1), lambda i: (i, 0))
_W1_SPEC = pl.BlockSpec((8, D), lambda i: (0, 0))
_SC_SPEC = pl.BlockSpec((8, 4), lambda i: (0, 0))
_GRID = (N + _BR - 1) // _BR
_BLEND_OUT = (jax.ShapeDtypeStruct((N, D), jnp.float32),
              jax.ShapeDtypeStruct((2, N, DH), jnp.float32))
_BLEND_OUT_SPECS = (_ROWS_SPEC, _SPLIT_SPEC)


def _blend_u(base, s2, t2, ss, st, w1_all, sc_all, ja, jb):
  return pl.pallas_call(
      functools.partial(_blend_u_body, ja, jb),
      out_shape=_BLEND_OUT,
      grid=(_GRID,),
      in_specs=[_ROWS_SPEC, _SPLIT_SPEC, _SPLIT_SPEC, _SEG_SPEC, _SEG_SPEC,
                _W1_SPEC, _SC_SPEC],
      out_specs=_BLEND_OUT_SPECS,
  )(base, s2, t2, ss, st, w1_all, sc_all)


def _blend_i(base, t2, st, w1_all, sc_all, ja, jb):
  return pl.pallas_call(
      functools.partial(_blend_i_body, ja, jb),
      out_shape=_BLEND_OUT,
      grid=(_GRID,),
      in_specs=[_ROWS_SPEC, _SPLIT_SPEC, _SEG_SPEC, _W1_SPEC, _SC_SPEC],
      out_specs=_BLEND_OUT_SPECS,
  )(base, t2, st, w1_all, sc_all)


def kernel(user_embedding, item_embedding, social_vals, ui_vals, iu_vals,
           low_W1, low_b1, low_W2, low_b2, ga_W1, ga_b1, ga_W2, ga_b2,
           social_idx, ui_idx, iu_idx):
  u0, i0 = user_embedding, item_embedding
  sd = social_idx[:, 0].astype(jnp.int32)
  ss = social_idx[:, 1].astype(jnp.int32)
  uid = ui_idx[:, 0].astype(jnp.int32)
  uis = ui_idx[:, 1].astype(jnp.int32)
  iud = iu_idx[:, 0].astype(jnp.int32)
  ius = iu_idx[:, 1].astype(jnp.int32)
  lowp = jnp.zeros((6, 8), jnp.float32)
  lowp = lowp.at[:, 0].set(low_W1[:, 0, 0])
  lowp = lowp.at[:, 1].set(low_b1[:, 0])
  lowp = lowp.at[:, 2].set(low_W2[:, 0, 0])
  lowp = lowp.at[:, 3].set(low_b2[:, 0])
  lowp = lowp.reshape(48)
  ga_w1 = ga_W1[:, :, 0]
  ga_sc = jnp.zeros((8, 4), jnp.float32)
  ga_sc = ga_sc.at[:, 0].set(ga_b1[:, 0])
  ga_sc = ga_sc.at[:, 1].set(ga_W2[:, 0, 0])
  ga_sc = ga_sc.at[:, 2].set(ga_b2[:, 0])

  lp = [lax.dynamic_slice(lowp, (8 * gl,), (8,)) for gl in range(6)]

  def split(x):
    return x.reshape(N, 2, DH).transpose(1, 0, 2)

  def pack_edges(s, d, v):
    vb = lax.bitcast_convert_type(v, jnp.int32)
    blk = lambda x: x.reshape(NS, CHUNKS, K)
    return jnp.stack([blk(s), blk(d), blk(vb)], axis=2).reshape(-1)

  ed_s = pack_edges(ss, sd, social_vals)
  ed_ui = pack_edges(uis, uid, ui_vals)
  ed_iu = pack_edges(ius, iud, iu_vals)
  u0s, i0s = split(u0), split(i0)

  u_soc1, s_s1 = _spmm(u0s, ed_s, lp[0])
  u_int1, s_ui1 = _spmm(i0s, ed_ui, lp[2])
  i_cust1, s_iu1 = _spmm(u0s, ed_iu, lp[4])
  u1, u1s = _blend_u(u0, u_soc1, u_int1,
                     s_s1.reshape(N, 1), s_ui1.reshape(N, 1),
                     ga_w1, ga_sc, 0, 1)
  i1, i1s = _blend_i(i0, i_cust1, s_iu1.reshape(N, 1), ga_w1, ga_sc, 4, 5)

  u_soc2, s_s2 = _spmm(u1s, ed_s, lp[1])
  u_int2, s_ui2 = _spmm(i1s, ed_ui, lp[3])
  i_cust2, s_iu2 = _spmm(u1s, ed_iu, lp[5])
  u2, _ = _blend_u(u1, u_soc2, u_int2,
                   s_s2.reshape(N, 1), s_ui2.reshape(N, 1),
                   ga_w1, ga_sc, 2, 3)
  i2, _ = _blend_i(i1, i_cust2, s_iu2.reshape(N, 1), ga_w1, ga_sc, 6, 7)

  final_user = jnp.concatenate([u0, u1, u2], axis=1)
  final_item = jnp.concatenate([i0, i1, i2], axis=1)
  return (final_user, final_item)
